# Initial kernel scaffold; baseline (speedup 1.0000x reference)
import jax
import jax.numpy as jnp
from jax.experimental import pallas as pl


def kernel(values_reduced, imposed_full, free_idx, conn):
    raise NotImplementedError("write your pallas kernel here")


# speedup vs baseline: 1.5519x; 1.5519x over previous
import jax
import jax.numpy as jnp
from jax import lax
from jax.experimental import pallas as pl
from jax.experimental.pallas import tpu as pltpu
from jax.experimental.pallas import tpu_sc as plsc


def _field_sc(values_reduced, imposed_full, free_idx, conn2d,
              n_nodes, n_free, n_idx_rows):
  info = plsc.get_sparse_core_info()
  nc, ns = info.num_cores, info.num_subcores
  nw = nc * ns

  C1 = 896
  n1 = -(-n_nodes // C1)
  s1 = -(-n1 // ns)
  r1_last = n_nodes - C1

  nb2 = -(-n_free // 128)
  s2 = -(-nb2 // ns)
  off2_last = n_free - 128

  CHUNK = 16
  rows_w = -(-n_idx_rows // nw)
  rows_w = -(-rows_w // CHUNK) * CHUNK
  base_last = n_idx_rows - rows_w
  steps = rows_w // CHUNK

  mesh = plsc.VectorSubcoreMesh(core_axis_name="c", subcore_axis_name="s")

  @pl.kernel(
      out_type=jax.ShapeDtypeStruct((n_idx_rows, 128, 3), jnp.float32),
      mesh=mesh,
      compiler_params=pltpu.CompilerParams(use_tc_tiling_on_sc=False),
      scratch_types=[
          pltpu.VMEM_SHARED((n_nodes, 8), jnp.float32),
          pltpu.VMEM((C1, 8), jnp.float32),
          pltpu.VMEM((1, 128), jnp.int32),
          pltpu.VMEM((128, 8), jnp.float32),
          pltpu.VMEM((CHUNK, 128), jnp.int32),
          pltpu.VMEM((CHUNK, 128, 8), jnp.float32),
      ],
  )
  def body(vals_hbm, imp_hbm, free_hbm, conn_hbm, out_hbm,
           table, stage, sidx, svals, gidx, grows):
    cid = lax.axis_index("c")
    sid = lax.axis_index("s")
    wid = sid * nc + cid

    @pl.loop(0, s1)
    def _copy(i):
      c = sid + i * ns
      r0 = jnp.minimum(c * C1, r1_last)
      pltpu.sync_copy(imp_hbm.at[pl.ds(r0, C1)], stage.at[:, pl.ds(0, 3)])
      pltpu.sync_copy(stage, table.at[pl.ds(r0, C1)])
    plsc.subcore_barrier()

    @pl.loop(0, s2)
    def _scatter(k):
      b = sid + k * ns
      off = jnp.minimum(b * 128, off2_last)
      pltpu.sync_copy(free_hbm.at[pl.ds(off, 128)], sidx.at[0])
      pltpu.sync_copy(vals_hbm.at[pl.ds(off, 128)], svals.at[:, pl.ds(0, 3)])
      pltpu.sync_copy(svals, table.at[sidx.at[0]])
    plsc.subcore_barrier()

    base = jnp.minimum(wid * rows_w, base_last)

    @pl.loop(0, steps)
    def _gather(t):
      r = base + t * CHUNK
      pltpu.sync_copy(conn_hbm.at[pl.ds(r, CHUNK)], gidx)
      for j in range(CHUNK):
        pltpu.sync_copy(table.at[gidx.at[j]], grows.at[j])
      pltpu.sync_copy(grows.at[:, :, pl.ds(0, 3)], out_hbm.at[pl.ds(r, CHUNK)])

  return body(values_reduced, imposed_full, free_idx, conn2d)


def kernel(values_reduced, imposed_full, free_idx, conn):
  n_nodes = imposed_full.shape[0]
  n_free = values_reduced.shape[0]
  n_elem, npe = conn.shape
  n_flat = n_elem * npe
  assert n_flat % 128 == 0
  n_idx_rows = n_flat // 128
  conn2d = conn.reshape(n_idx_rows, 128)
  out = _field_sc(values_reduced, imposed_full, free_idx, conn2d,
                  n_nodes, n_free, n_idx_rows)
  return out.reshape(n_elem, npe, 3)
